# baseline (device time: 66206 ns/iter reference)
import jax
import jax.numpy as jnp
from jax import lax
from jax.experimental import pallas as pl
from jax.experimental.pallas import tpu as pltpu

B = 2
S = 1024
S_HALF = 512
K = 1024
N = 2048

CS = 128
NCB = S_HALF // CS
NCHUNK = B * NCB

_DN = (((0,), (0,)), ((), ()))


def kernel(O, Wo):
    O2T = O.reshape(B, S, K).transpose(0, 2, 1)

    def body(o_ref, w_ref, out_ref, send_buf, recv_buf, send_sems, recv_sems):
        my_x = lax.axis_index("x")
        my_y = lax.axis_index("y")
        my_z = lax.axis_index("z")
        peer = (1 - my_x, my_y, my_z)

        barrier_sem = pltpu.get_barrier_semaphore()
        pl.semaphore_signal(
            barrier_sem, inc=1, device_id=peer,
            device_id_type=pl.DeviceIdType.MESH,
        )
        pl.semaphore_wait(barrier_sem, 1)

        w = w_ref[...].astype(jnp.bfloat16)

        def run(peer_start, my_start):
            rdmas = []
            for idx in range(NCHUNK):
                b, c = divmod(idx, NCB)
                s0 = peer_start + c * CS
                lhsT = o_ref[b, :, s0:s0 + CS].astype(jnp.bfloat16)
                send_buf[idx, :, :] = lax.dot_general(
                    lhsT, w, _DN, preferred_element_type=jnp.float32
                ).astype(jnp.bfloat16)
                rdma = pltpu.make_async_remote_copy(
                    src_ref=send_buf.at[idx],
                    dst_ref=recv_buf.at[idx],
                    send_sem=send_sems.at[idx],
                    recv_sem=recv_sems.at[idx],
                    device_id=peer,
                    device_id_type=pl.DeviceIdType.MESH,
                )
                rdma.start()
                rdmas.append(rdma)

            for b in range(B):
                lhsT = o_ref[b, :, my_start:my_start + S_HALF].astype(
                    jnp.bfloat16
                )
                out_ref[b, :, :] = lax.dot_general(
                    lhsT, w, _DN, preferred_element_type=jnp.float32
                )

            for idx, rdma in enumerate(rdmas):
                b, c = divmod(idx, NCB)
                rdma.wait_send()
                rdma.wait_recv()
                sl = pl.ds(c * CS, CS)
                out_ref[b, sl, :] = out_ref[b, sl, :] + recv_buf[idx].astype(
                    jnp.float32
                )

        @pl.when(my_x == 0)
        def _():
            run(S_HALF, 0)

        @pl.when(my_x == 1)
        def _():
            run(0, S_HALF)

    return pl.pallas_call(
        body,
        out_shape=jax.ShapeDtypeStruct((B, S_HALF, N), jnp.float32),
        in_specs=[
            pl.BlockSpec(memory_space=pltpu.VMEM),
            pl.BlockSpec(memory_space=pltpu.VMEM),
        ],
        out_specs=pl.BlockSpec(memory_space=pltpu.VMEM),
        scratch_shapes=[
            pltpu.VMEM((NCHUNK, CS, N), jnp.bfloat16),
            pltpu.VMEM((NCHUNK, CS, N), jnp.bfloat16),
            pltpu.SemaphoreType.DMA((NCHUNK,)),
            pltpu.SemaphoreType.DMA((NCHUNK,)),
        ],
        compiler_params=pltpu.CompilerParams(
            collective_id=0, vmem_limit_bytes=100 * 1024 * 1024
        ),
    )(O2T, Wo)


# device time: 62766 ns/iter; 1.0548x vs baseline; 1.0548x over previous
import jax
import jax.numpy as jnp
from jax import lax
from jax.experimental import pallas as pl
from jax.experimental.pallas import tpu as pltpu

B = 2
S = 1024
S_HALF = 512
K = 1024
N = 2048

CS = 128
NCB = S_HALF // CS
NCHUNK = B * NCB

_DN = (((0,), (0,)), ((), ()))


def kernel(O, Wo):
    O2T = O.reshape(B, S, K).transpose(0, 2, 1)

    def body(o_ref, w_ref, out_ref, st_buf, send_buf, recv_buf,
             send_sems, recv_sems, st_sems):
        my_x = lax.axis_index("x")
        my_y = lax.axis_index("y")
        my_z = lax.axis_index("z")
        peer = (1 - my_x, my_y, my_z)

        barrier_sem = pltpu.get_barrier_semaphore()
        pl.semaphore_signal(
            barrier_sem, inc=1, device_id=peer,
            device_id_type=pl.DeviceIdType.MESH,
        )
        pl.semaphore_wait(barrier_sem, 1)

        w = w_ref[...].astype(jnp.bfloat16)

        def run(peer_start, my_start):
            rdmas = []
            for idx in range(NCHUNK):
                b, c = divmod(idx, NCB)
                s0 = peer_start + c * CS
                lhsT = o_ref[b, :, s0:s0 + CS].astype(jnp.bfloat16)
                send_buf[idx, :, :] = lax.dot_general(
                    lhsT, w, _DN, preferred_element_type=jnp.float32
                ).astype(jnp.bfloat16)
                rdma = pltpu.make_async_remote_copy(
                    src_ref=send_buf.at[idx],
                    dst_ref=recv_buf.at[idx],
                    send_sem=send_sems.at[idx],
                    recv_sem=recv_sems.at[idx],
                    device_id=peer,
                    device_id_type=pl.DeviceIdType.MESH,
                )
                rdma.start()
                rdmas.append(rdma)

            st_cps = []
            for idx, rdma in enumerate(rdmas):
                b, c = divmod(idx, NCB)
                s0 = my_start + c * CS
                lhsT = o_ref[b, :, s0:s0 + CS].astype(jnp.bfloat16)
                mine = lax.dot_general(
                    lhsT, w, _DN, preferred_element_type=jnp.float32
                )
                rdma.wait_send()
                rdma.wait_recv()
                st_buf[idx, :, :] = (
                    mine + recv_buf[idx].astype(jnp.float32)
                ).astype(jnp.bfloat16)
                cp = pltpu.make_async_copy(
                    st_buf.at[idx],
                    out_ref.at[b, pl.ds(c * CS, CS), :],
                    st_sems.at[idx],
                )
                cp.start()
                st_cps.append(cp)
            for cp in st_cps:
                cp.wait()

        @pl.when(my_x == 0)
        def _():
            run(S_HALF, 0)

        @pl.when(my_x == 1)
        def _():
            run(0, S_HALF)

    return pl.pallas_call(
        body,
        out_shape=jax.ShapeDtypeStruct((B, S_HALF, N), jnp.bfloat16),
        in_specs=[
            pl.BlockSpec(memory_space=pltpu.VMEM),
            pl.BlockSpec(memory_space=pltpu.VMEM),
        ],
        out_specs=pl.BlockSpec(memory_space=pltpu.MemorySpace.HBM),
        scratch_shapes=[
            pltpu.VMEM((NCHUNK, CS, N), jnp.bfloat16),
            pltpu.VMEM((NCHUNK, CS, N), jnp.bfloat16),
            pltpu.VMEM((NCHUNK, CS, N), jnp.bfloat16),
            pltpu.SemaphoreType.DMA((NCHUNK,)),
            pltpu.SemaphoreType.DMA((NCHUNK,)),
            pltpu.SemaphoreType.DMA((NCHUNK,)),
        ],
        compiler_params=pltpu.CompilerParams(
            collective_id=0, vmem_limit_bytes=100 * 1024 * 1024
        ),
    )(O2T, Wo)
